# baseline (speedup 1.0000x reference)
import functools

import jax
import jax.numpy as jnp
from jax import lax
from jax.experimental import pallas as pl
from jax.experimental.pallas import tpu as pltpu

N = 10000
NFEAT = 128
NHID = 64
NCLASS = 64
NHEADS = 4
ALPHA = 0.2

NPAD = 10240
BN = 256


def _elu(z):
    return jnp.where(z > 0, z, jnp.exp(z) - 1.0)


def _mm1_body(x_ref, wcat_ref, wa_ref, h_ref, f_ref):
    xb = x_ref[...]
    hb = jnp.dot(xb, wcat_ref[...], preferred_element_type=jnp.float32)
    h_ref[0, :, :] = hb[:, :128]
    h_ref[1, :, :] = hb[:, 128:]
    f_ref[...] = jnp.dot(xb, wa_ref[...], preferred_element_type=jnp.float32)


def _mm1(xpad, wcat, wa):
    return pl.pallas_call(
        _mm1_body,
        grid=(NPAD // BN,),
        in_specs=[
            pl.BlockSpec((BN, NFEAT), lambda i: (i, 0)),
            pl.BlockSpec((NFEAT, 2 * 128), lambda i: (0, 0)),
            pl.BlockSpec((NFEAT, 8), lambda i: (0, 0)),
        ],
        out_specs=[
            pl.BlockSpec((2, BN, 128), lambda i: (0, i, 0)),
            pl.BlockSpec((BN, 8), lambda i: (i, 0)),
        ],
        out_shape=[
            jax.ShapeDtypeStruct((2, NPAD, 128), jnp.float32),
            jax.ShapeDtypeStruct((NPAD, 8), jnp.float32),
        ],
    )(xpad, wcat, wa)


def _mm2_body(acc_ref, wout_ref, wa2_ref, h2_ref, f2_ref):
    parts = []
    for i in range(NHEADS):
        c, hh = divmod(i, 2)
        v = acc_ref[c, :, hh * 64:(hh + 1) * 64]
        rs = acc_ref[c, :, 128 + hh][:, None]
        parts.append(_elu(v / (rs + 1e-16)))
    x2b = jnp.concatenate(parts, axis=1)
    h2_ref[...] = jnp.dot(x2b, wout_ref[...], preferred_element_type=jnp.float32)
    f2_ref[...] = jnp.dot(x2b, wa2_ref[...], preferred_element_type=jnp.float32)


def _mm2(acc1, w_out, wa2):
    return pl.pallas_call(
        _mm2_body,
        grid=(NPAD // BN,),
        in_specs=[
            pl.BlockSpec((2, BN, 144), lambda i: (0, i, 0)),
            pl.BlockSpec((NHEADS * NHID, NCLASS), lambda i: (0, 0)),
            pl.BlockSpec((NHEADS * NHID, 8), lambda i: (0, 0)),
        ],
        out_specs=[
            pl.BlockSpec((BN, NCLASS), lambda i: (i, 0)),
            pl.BlockSpec((BN, 8), lambda i: (i, 0)),
        ],
        out_shape=[
            jax.ShapeDtypeStruct((NPAD, NCLASS), jnp.float32),
            jax.ShapeDtypeStruct((NPAD, 8), jnp.float32),
        ],
    )(acc1, w_out, wa2)


def _fin_body(acc_ref, out_ref):
    s = acc_ref[0, :, :64] + acc_ref[1, :, :64]
    rs = (acc_ref[0, :, 64] + acc_ref[1, :, 64])[:, None]
    out_ref[...] = _elu(s / (rs + 1e-16))


def _fin(acc2):
    return pl.pallas_call(
        _fin_body,
        grid=(NPAD // BN,),
        in_specs=[pl.BlockSpec((2, BN, 80), lambda i: (0, i, 0))],
        out_specs=pl.BlockSpec((BN, NCLASS), lambda i: (i, 0)),
        out_shape=jax.ShapeDtypeStruct((NPAD, NCLASS), jnp.float32),
    )(acc2)


def _leaky(z):
    return jnp.where(z >= 0, z, ALPHA * z)


def _edge1_jax(src, dst, h, f):
    w = jnp.exp(-_leaky(f[src, 0:4] + f[dst, 4:8]))
    accs = []
    for c in range(2):
        hd = h[c][dst]
        scaled = jnp.concatenate(
            [hd[:, :64] * w[:, 2 * c, None], hd[:, 64:] * w[:, 2 * c + 1, None],
             w[:, 2 * c:2 * c + 2], jnp.zeros((src.shape[0], 14), jnp.float32)],
            axis=1)
        accs.append(jax.ops.segment_sum(scaled, src, num_segments=NPAD))
    return jnp.stack(accs)


def _edge2_jax(src, dst, h2, f2):
    w = jnp.exp(-_leaky(f2[src, 0] + f2[dst, 1]))
    scaled = jnp.concatenate(
        [h2[dst] * w[:, None], w[:, None],
         jnp.zeros((src.shape[0], 15), jnp.float32)], axis=1)
    half = src.shape[0] // 2
    a0 = jax.ops.segment_sum(scaled[:half], src[:half], num_segments=NPAD)
    a1 = jax.ops.segment_sum(scaled[half:], src[half:], num_segments=NPAD)
    return jnp.stack([a0, a1])


def kernel(adj, x, W, a, W_out, a_out):
    src = adj[0].astype(jnp.int32)
    dst = adj[1].astype(jnp.int32)

    wcat = jnp.transpose(W, (1, 0, 2)).reshape(NFEAT, NHEADS * NHID)
    a_src = a[:, :NHID]
    a_dst = a[:, NHID:]
    blk_s = jax.scipy.linalg.block_diag(*[a_src[i][:, None] for i in range(NHEADS)])
    blk_d = jax.scipy.linalg.block_diag(*[a_dst[i][:, None] for i in range(NHEADS)])
    wa = jnp.concatenate([wcat @ blk_s, wcat @ blk_d], axis=1)
    wa2 = jnp.concatenate(
        [(W_out @ a_out[:NCLASS])[:, None], (W_out @ a_out[NCLASS:])[:, None],
         jnp.zeros((NHEADS * NHID, 6), jnp.float32)], axis=1)

    xpad = jnp.pad(x, ((0, NPAD - N), (0, 0)))

    h, f = _mm1(xpad, wcat, wa)
    acc1 = _edge1_jax(src, dst, h, f)
    h2, f2 = _mm2(acc1, W_out, wa2)
    acc2 = _edge2_jax(src, dst, h2, f2)
    out = _fin(acc2)
    return out[:N]


# speedup vs baseline: 22.6334x; 22.6334x over previous
import functools

import jax
import jax.numpy as jnp
from jax import lax
from jax.experimental import pallas as pl
from jax.experimental.pallas import tpu as pltpu
from jax.experimental.pallas import tpu_sc as plsc

N = 10000
NFEAT = 128
NHID = 64
NCLASS = 64
NHEADS = 4
ALPHA = 0.2

NPAD = 10240
BN = 256


def _elu(z):
    return jnp.where(z > 0, z, jnp.exp(z) - 1.0)


def _mm1_body(x_ref, wcat_ref, wa_ref, h_ref, f_ref):
    xb = x_ref[...]
    hb = jnp.dot(xb, wcat_ref[...], preferred_element_type=jnp.float32)
    h_ref[0, :, :] = hb[:, :128]
    h_ref[1, :, :] = hb[:, 128:]
    f_ref[...] = jnp.dot(xb, wa_ref[...], preferred_element_type=jnp.float32)


def _mm1(xpad, wcat, wa):
    return pl.pallas_call(
        _mm1_body,
        grid=(NPAD // BN,),
        in_specs=[
            pl.BlockSpec((BN, NFEAT), lambda i: (i, 0)),
            pl.BlockSpec((NFEAT, 2 * 128), lambda i: (0, 0)),
            pl.BlockSpec((NFEAT, 8), lambda i: (0, 0)),
        ],
        out_specs=[
            pl.BlockSpec((2, BN, 128), lambda i: (0, i, 0)),
            pl.BlockSpec((BN, 8), lambda i: (i, 0)),
        ],
        out_shape=[
            jax.ShapeDtypeStruct((2, NPAD, 128), jnp.float32),
            jax.ShapeDtypeStruct((NPAD, 8), jnp.float32),
        ],
    )(xpad, wcat, wa)


def _mm2_body(acc_ref, wout_ref, wa2_ref, h2_ref, f2_ref):
    parts = []
    for i in range(NHEADS):
        c, hh = divmod(i, 2)
        v = acc_ref[c, :, hh * 64:(hh + 1) * 64]
        rs = acc_ref[c, :, 128 + hh][:, None]
        parts.append(_elu(v / (rs + 1e-16)))
    x2b = jnp.concatenate(parts, axis=1)
    h2_ref[...] = jnp.dot(x2b, wout_ref[...], preferred_element_type=jnp.float32)
    f2_ref[...] = jnp.dot(x2b, wa2_ref[...], preferred_element_type=jnp.float32)


def _mm2(acc1, w_out, wa2):
    return pl.pallas_call(
        _mm2_body,
        grid=(NPAD // BN,),
        in_specs=[
            pl.BlockSpec((2, BN, 144), lambda i: (0, i, 0)),
            pl.BlockSpec((NHEADS * NHID, NCLASS), lambda i: (0, 0)),
            pl.BlockSpec((NHEADS * NHID, 8), lambda i: (0, 0)),
        ],
        out_specs=[
            pl.BlockSpec((BN, NCLASS), lambda i: (i, 0)),
            pl.BlockSpec((BN, 8), lambda i: (i, 0)),
        ],
        out_shape=[
            jax.ShapeDtypeStruct((NPAD, NCLASS), jnp.float32),
            jax.ShapeDtypeStruct((NPAD, 8), jnp.float32),
        ],
    )(acc1, w_out, wa2)


def _fin_body(acc_ref, out_ref):
    s = acc_ref[0, :, :64] + acc_ref[1, :, :64]
    rs = (acc_ref[0, :, 64] + acc_ref[1, :, 64])[:, None]
    out_ref[...] = _elu(s / (rs + 1e-16))


def _fin(acc2):
    return pl.pallas_call(
        _fin_body,
        grid=(NPAD // BN,),
        in_specs=[pl.BlockSpec((2, BN, 80), lambda i: (0, i, 0))],
        out_specs=pl.BlockSpec((BN, NCLASS), lambda i: (i, 0)),
        out_shape=jax.ShapeDtypeStruct((NPAD, NCLASS), jnp.float32),
    )(acc2)


def _leaky(z):
    return jnp.where(z >= 0, z, ALPHA * z)


EPAD = 322560
CHUNK = 96
NSUB = 16
ROWS_PER_SUB = NPAD // NSUB
WCOPY = 80


def _splat_i32(x):
    return jnp.zeros((16,), jnp.int32) + x


def _zero_vmem_rows(ref, nrows, ngroups):
    def zrow(r, _):
        for g in range(ngroups):
            ref[r, pl.ds(g * 16, 16)] = jnp.zeros((16,), jnp.float32)
        return 0
    lax.fori_loop(0, nrows, zrow, 0)


def _edge_kernel1(src_hbm, dst_hbm, h_hbm, f_hbm, out_hbm,
                  acc_sh, sidx_v, didx_v, gidx_v, fs_v, fd_v, w_v,
                  rows_v, outb_v, sem, semf):
    c = lax.axis_index("c")
    s = lax.axis_index("s")

    _zero_vmem_rows(outb_v, CHUNK, 9)
    for k in range(ROWS_PER_SUB // WCOPY):
        pltpu.sync_copy(outb_v.at[pl.ds(0, WCOPY)],
                        acc_sh.at[pl.ds(s * ROWS_PER_SUB + k * WCOPY, WCOPY)])
    plsc.subcore_barrier()

    ep_tile = EPAD // NSUB
    base_t = s * ep_tile

    def chunk_body(g, _):
        base = base_t + g * CHUNK
        pltpu.sync_copy(src_hbm.at[pl.ds(base, CHUNK)], sidx_v)
        pltpu.sync_copy(dst_hbm.at[pl.ds(base, CHUNK)], didx_v)
        for b in range(CHUNK // 16):
            gidx_v[pl.ds(b * 16, 16)] = didx_v[pl.ds(b * 16, 16)] + c * NPAD
        cp_h = pltpu.async_copy(h_hbm.at[gidx_v], rows_v, sem)
        cp_fs = pltpu.async_copy(f_hbm.at[sidx_v], fs_v, semf)
        cp_fd = pltpu.async_copy(f_hbm.at[didx_v], fd_v, semf)
        cp_fs.wait()
        cp_fd.wait()
        lanes = lax.iota(jnp.int32, 16)
        for b in range(CHUNK // 16):
            for hh in range(2):
                head = c * 2 + hh
                fs = plsc.load_gather(fs_v, [lanes + b * 16, _splat_i32(head)])
                fd = plsc.load_gather(fd_v, [lanes + b * 16, _splat_i32(head + 4)])
                z = fs + fd
                w = jnp.exp(-jnp.where(z >= 0, z, ALPHA * z))
                w_v[pl.ds(hh * CHUNK + b * 16, 16)] = w
        cp_h.wait()

        def edge_body(e, _):
            wv0 = plsc.load_gather(w_v, [_splat_i32(e)])
            wv1 = plsc.load_gather(w_v, [_splat_i32(e + CHUNK)])
            for gg in range(4):
                outb_v[e, pl.ds(gg * 16, 16)] = rows_v[e, pl.ds(gg * 16, 16)] * wv0
            for gg in range(4, 8):
                outb_v[e, pl.ds(gg * 16, 16)] = rows_v[e, pl.ds(gg * 16, 16)] * wv1
            iv = lax.iota(jnp.int32, 16)
            tail = jnp.where(iv == 0, wv0, jnp.where(iv == 1, wv1, 0.0))
            outb_v[e, pl.ds(128, 16)] = tail
            return 0
        lax.fori_loop(0, CHUNK, edge_body, 0)
        pltpu.sync_copy(outb_v, acc_sh.at[sidx_v], add=True)
        return 0
    lax.fori_loop(0, ep_tile // CHUNK, chunk_body, 0)

    plsc.subcore_barrier()
    for k in range(ROWS_PER_SUB // WCOPY):
        r0 = s * ROWS_PER_SUB + k * WCOPY
        pltpu.sync_copy(acc_sh.at[pl.ds(r0, WCOPY)], out_hbm.at[c, pl.ds(r0, WCOPY)])


def _edge1_sc(src, dst, h2c, f):
    mesh = plsc.VectorSubcoreMesh(core_axis_name="c", subcore_axis_name="s",
                                  num_cores=2, num_subcores=NSUB)
    run = pl.kernel(
        _edge_kernel1,
        mesh=mesh,
        compiler_params=pltpu.CompilerParams(needs_layout_passes=False,
                                             use_tc_tiling_on_sc=False),
        out_type=jax.ShapeDtypeStruct((2, NPAD, 144), jnp.float32),
        scratch_types=[
            pltpu.VMEM_SHARED((NPAD, 144), jnp.float32),
            pltpu.VMEM((CHUNK,), jnp.int32),
            pltpu.VMEM((CHUNK,), jnp.int32),
            pltpu.VMEM((CHUNK,), jnp.int32),
            pltpu.VMEM((CHUNK, 8), jnp.float32),
            pltpu.VMEM((CHUNK, 8), jnp.float32),
            pltpu.VMEM((2 * CHUNK,), jnp.float32),
            pltpu.VMEM((CHUNK, 128), jnp.float32),
            pltpu.VMEM((CHUNK, 144), jnp.float32),
            pltpu.SemaphoreType.DMA,
            pltpu.SemaphoreType.DMA,
        ],
    )
    return run(src, dst, h2c, f)


def _edge_kernel2(src_hbm, dst_hbm, h_hbm, f_hbm, out_hbm,
                  acc_sh, f_v, sidx_v, didx_v, w_v, rows_v, outb_v, sem):
    c = lax.axis_index("c")
    s = lax.axis_index("s")

    _zero_vmem_rows(outb_v, CHUNK, 5)
    for k in range(ROWS_PER_SUB // WCOPY):
        pltpu.sync_copy(outb_v.at[pl.ds(0, WCOPY)],
                        acc_sh.at[pl.ds(s * ROWS_PER_SUB + k * WCOPY, WCOPY)])
    plsc.subcore_barrier()

    pltpu.sync_copy(f_hbm, f_v)

    ep_w = EPAD // (2 * NSUB)
    base_t = c * (EPAD // 2) + s * ep_w

    def chunk_body(g, _):
        base = base_t + g * CHUNK
        pltpu.sync_copy(src_hbm.at[pl.ds(base, CHUNK)], sidx_v)
        pltpu.sync_copy(dst_hbm.at[pl.ds(base, CHUNK)], didx_v)
        cp_h = pltpu.async_copy(h_hbm.at[didx_v], rows_v, sem)
        for b in range(CHUNK // 16):
            sv = sidx_v[pl.ds(b * 16, 16)]
            dv = didx_v[pl.ds(b * 16, 16)]
            fs = plsc.load_gather(f_v, [sv * 2])
            fd = plsc.load_gather(f_v, [dv * 2 + 1])
            z = fs + fd
            w_v[pl.ds(b * 16, 16)] = jnp.exp(-jnp.where(z >= 0, z, ALPHA * z))
        cp_h.wait()

        def edge_body(e, _):
            wv = plsc.load_gather(w_v, [_splat_i32(e)])
            for gg in range(4):
                outb_v[e, pl.ds(gg * 16, 16)] = rows_v[e, pl.ds(gg * 16, 16)] * wv
            iv = lax.iota(jnp.int32, 16)
            outb_v[e, pl.ds(64, 16)] = jnp.where(iv == 0, wv, 0.0)
            return 0
        lax.fori_loop(0, CHUNK, edge_body, 0)
        pltpu.sync_copy(outb_v, acc_sh.at[sidx_v], add=True)
        return 0
    lax.fori_loop(0, ep_w // CHUNK, chunk_body, 0)

    plsc.subcore_barrier()
    for k in range(ROWS_PER_SUB // WCOPY):
        r0 = s * ROWS_PER_SUB + k * WCOPY
        pltpu.sync_copy(acc_sh.at[pl.ds(r0, WCOPY)], out_hbm.at[c, pl.ds(r0, WCOPY)])


def _edge2_sc(src, dst, h2, f2_flat):
    mesh = plsc.VectorSubcoreMesh(core_axis_name="c", subcore_axis_name="s",
                                  num_cores=2, num_subcores=NSUB)
    run = pl.kernel(
        _edge_kernel2,
        mesh=mesh,
        compiler_params=pltpu.CompilerParams(needs_layout_passes=False,
                                             use_tc_tiling_on_sc=False),
        out_type=jax.ShapeDtypeStruct((2, NPAD, 80), jnp.float32),
        scratch_types=[
            pltpu.VMEM_SHARED((NPAD, 80), jnp.float32),
            pltpu.VMEM((NPAD * 2,), jnp.float32),
            pltpu.VMEM((CHUNK,), jnp.int32),
            pltpu.VMEM((CHUNK,), jnp.int32),
            pltpu.VMEM((CHUNK,), jnp.float32),
            pltpu.VMEM((CHUNK, 64), jnp.float32),
            pltpu.VMEM((CHUNK, 80), jnp.float32),
            pltpu.SemaphoreType.DMA,
        ],
    )
    return run(src, dst, h2, f2_flat)


def kernel(adj, x, W, a, W_out, a_out):
    src = adj[0].astype(jnp.int32)
    dst = adj[1].astype(jnp.int32)

    wcat = jnp.transpose(W, (1, 0, 2)).reshape(NFEAT, NHEADS * NHID)
    a_src = a[:, :NHID]
    a_dst = a[:, NHID:]
    blk_s = jax.scipy.linalg.block_diag(*[a_src[i][:, None] for i in range(NHEADS)])
    blk_d = jax.scipy.linalg.block_diag(*[a_dst[i][:, None] for i in range(NHEADS)])
    wa = jnp.concatenate([wcat @ blk_s, wcat @ blk_d], axis=1)
    wa2 = jnp.concatenate(
        [(W_out @ a_out[:NCLASS])[:, None], (W_out @ a_out[NCLASS:])[:, None],
         jnp.zeros((NHEADS * NHID, 6), jnp.float32)], axis=1)

    xpad = jnp.pad(x, ((0, NPAD - N), (0, 0)))
    src_p = jnp.pad(src, (0, EPAD - src.shape[0]), constant_values=N)
    dst_p = jnp.pad(dst, (0, EPAD - dst.shape[0]), constant_values=0)

    h, f = _mm1(xpad, wcat, wa)
    acc1 = _edge1_sc(src_p, dst_p, h.reshape(2 * NPAD, 128), f)
    h2, f2 = _mm2(acc1, W_out, wa2)
    acc2 = _edge2_sc(src_p, dst_p, h2, f2[:, :2].reshape(-1))
    out = _fin(acc2)
    return out[:N]


# speedup vs baseline: 46.8434x; 2.0697x over previous
import functools

import jax
import jax.numpy as jnp
from jax import lax
from jax.experimental import pallas as pl
from jax.experimental.pallas import tpu as pltpu
from jax.experimental.pallas import tpu_sc as plsc

N = 10000
NFEAT = 128
NHID = 64
NCLASS = 64
NHEADS = 4
ALPHA = 0.2

NPAD = 10240
BN = 256


def _elu(z):
    return jnp.where(z > 0, z, jnp.exp(z) - 1.0)


def _mm1_body(x_ref, wcat_ref, wa_ref, h_ref, f_ref):
    xb = x_ref[...]
    hb = jnp.dot(xb, wcat_ref[...], preferred_element_type=jnp.float32)
    h_ref[0, :, :] = hb[:, :128]
    h_ref[1, :, :] = hb[:, 128:]
    f_ref[...] = jnp.dot(xb, wa_ref[...], preferred_element_type=jnp.float32)


def _mm1(xpad, wcat, wa):
    return pl.pallas_call(
        _mm1_body,
        grid=(NPAD // BN,),
        in_specs=[
            pl.BlockSpec((BN, NFEAT), lambda i: (i, 0)),
            pl.BlockSpec((NFEAT, 2 * 128), lambda i: (0, 0)),
            pl.BlockSpec((NFEAT, 8), lambda i: (0, 0)),
        ],
        out_specs=[
            pl.BlockSpec((2, BN, 128), lambda i: (0, i, 0)),
            pl.BlockSpec((BN, 8), lambda i: (i, 0)),
        ],
        out_shape=[
            jax.ShapeDtypeStruct((2, NPAD, 128), jnp.float32),
            jax.ShapeDtypeStruct((NPAD, 8), jnp.float32),
        ],
    )(xpad, wcat, wa)


def _mm2_body(acc_ref, wout_ref, wa2_ref, h2_ref, f2_ref):
    parts = []
    for i in range(NHEADS):
        c, hh = divmod(i, 2)
        v = acc_ref[c, :, hh * 64:(hh + 1) * 64]
        rs = acc_ref[c, :, 128 + hh][:, None]
        parts.append(_elu(v / (rs + 1e-16)))
    x2b = jnp.concatenate(parts, axis=1)
    h2_ref[...] = jnp.dot(x2b, wout_ref[...], preferred_element_type=jnp.float32)
    f2_ref[...] = jnp.dot(x2b, wa2_ref[...], preferred_element_type=jnp.float32)


def _mm2(acc1, w_out, wa2):
    return pl.pallas_call(
        _mm2_body,
        grid=(NPAD // BN,),
        in_specs=[
            pl.BlockSpec((2, BN, 144), lambda i: (0, i, 0)),
            pl.BlockSpec((NHEADS * NHID, NCLASS), lambda i: (0, 0)),
            pl.BlockSpec((NHEADS * NHID, 8), lambda i: (0, 0)),
        ],
        out_specs=[
            pl.BlockSpec((BN, NCLASS), lambda i: (i, 0)),
            pl.BlockSpec((BN, 8), lambda i: (i, 0)),
        ],
        out_shape=[
            jax.ShapeDtypeStruct((NPAD, NCLASS), jnp.float32),
            jax.ShapeDtypeStruct((NPAD, 8), jnp.float32),
        ],
    )(acc1, w_out, wa2)


def _fin_body(acc_ref, out_ref):
    s = acc_ref[0, :, :64] + acc_ref[1, :, :64]
    rs = (acc_ref[0, :, 64] + acc_ref[1, :, 64])[:, None]
    out_ref[...] = _elu(s / (rs + 1e-16))


def _fin(acc2):
    return pl.pallas_call(
        _fin_body,
        grid=(NPAD // BN,),
        in_specs=[pl.BlockSpec((2, BN, 80), lambda i: (0, i, 0))],
        out_specs=pl.BlockSpec((BN, NCLASS), lambda i: (i, 0)),
        out_shape=jax.ShapeDtypeStruct((NPAD, NCLASS), jnp.float32),
    )(acc2)


def _leaky(z):
    return jnp.where(z >= 0, z, ALPHA * z)


EPAD = 322560
CHUNK = 80
NSUB = 16
ROWS_PER_SUB = NPAD // NSUB
WCOPY = 80
NBLK = CHUNK // 16


def _splat_i32(x):
    return jnp.zeros((16,), jnp.int32) + x


def _zero_vmem_rows(ref, nrows, ngroups):
    def zrow(r, _):
        for g in range(ngroups):
            ref[r, pl.ds(g * 16, 16)] = jnp.zeros((16,), jnp.float32)
        return 0
    lax.fori_loop(0, nrows, zrow, 0)


def _edge_kernel1(src_hbm, dst_hbm, h_hbm, f_hbm, out_hbm,
                  acc_sh, sidx0, sidx1, didx0, didx1, gidx0, gidx1,
                  fs0, fs1, fd0, fd1, w_v, rows0, rows1, outb_v,
                  semg0, semg1):
    c = lax.axis_index("c")
    s = lax.axis_index("s")
    sidx = (sidx0, sidx1)
    didx = (didx0, didx1)
    gidx = (gidx0, gidx1)
    fsb = (fs0, fs1)
    fdb = (fd0, fd1)
    rows = (rows0, rows1)
    semg = (semg0, semg1)

    _zero_vmem_rows(outb_v, CHUNK, 9)
    for k in range(ROWS_PER_SUB // WCOPY):
        pltpu.sync_copy(outb_v.at[pl.ds(0, WCOPY)],
                        acc_sh.at[pl.ds(s * ROWS_PER_SUB + k * WCOPY, WCOPY)])
    plsc.subcore_barrier()

    ep_tile = EPAD // NSUB
    base_t = s * ep_tile
    nch = ep_tile // CHUNK

    def fire(b, g):
        base = base_t + g * CHUNK
        pltpu.sync_copy(src_hbm.at[pl.ds(base, CHUNK)], sidx[b])
        pltpu.sync_copy(dst_hbm.at[pl.ds(base, CHUNK)], didx[b])
        for blk in range(NBLK):
            gidx[b][pl.ds(blk * 16, 16)] = didx[b][pl.ds(blk * 16, 16)] + c * NPAD
        pltpu.async_copy(h_hbm.at[gidx[b]], rows[b], semg[b])
        pltpu.async_copy(f_hbm.at[sidx[b]], fsb[b], semg[b])
        pltpu.async_copy(f_hbm.at[didx[b]], fdb[b], semg[b])

    def drain(b):
        pltpu.make_async_copy(h_hbm.at[gidx[b]], rows[b], semg[b]).wait()
        pltpu.make_async_copy(f_hbm.at[sidx[b]], fsb[b], semg[b]).wait()
        pltpu.make_async_copy(f_hbm.at[didx[b]], fdb[b], semg[b]).wait()

    def process(b):
        lanes = lax.iota(jnp.int32, 16)
        for blk in range(NBLK):
            for hh in range(2):
                head = c * 2 + hh
                fs = plsc.load_gather(fsb[b], [lanes + blk * 16, _splat_i32(head)])
                fd = plsc.load_gather(fdb[b], [lanes + blk * 16, _splat_i32(head + 4)])
                z = fs + fd
                w = jnp.exp(-jnp.where(z >= 0, z, ALPHA * z))
                w_v[pl.ds(hh * CHUNK + blk * 16, 16)] = w

        @plsc.parallel_loop(0, CHUNK, unroll=4)
        def _scale(e):
            wv0 = plsc.load_gather(w_v, [_splat_i32(e)])
            wv1 = plsc.load_gather(w_v, [_splat_i32(e + CHUNK)])
            for gg in range(4):
                outb_v[e, pl.ds(gg * 16, 16)] = rows[b][e, pl.ds(gg * 16, 16)] * wv0
            for gg in range(4, 8):
                outb_v[e, pl.ds(gg * 16, 16)] = rows[b][e, pl.ds(gg * 16, 16)] * wv1

        for blk in range(NBLK):
            ev = lanes + blk * 16
            plsc.store_scatter(outb_v, [ev, _splat_i32(128)],
                               w_v[pl.ds(blk * 16, 16)])
            plsc.store_scatter(outb_v, [ev, _splat_i32(129)],
                               w_v[pl.ds(CHUNK + blk * 16, 16)])
        pltpu.sync_copy(outb_v, acc_sh.at[sidx[b]], add=True)

    fire(0, 0)

    def outer(go, _):
        for b in range(2):
            g = go * 2 + b

            @pl.when(g + 1 < nch)
            def _():
                fire(1 - b, g + 1)
            drain(b)
            process(b)
        return 0
    lax.fori_loop(0, nch // 2, outer, 0)

    plsc.subcore_barrier()
    for k in range(ROWS_PER_SUB // WCOPY):
        r0 = s * ROWS_PER_SUB + k * WCOPY
        pltpu.sync_copy(acc_sh.at[pl.ds(r0, WCOPY)], out_hbm.at[c, pl.ds(r0, WCOPY)])


def _edge1_sc(src, dst, h2c, f):
    mesh = plsc.VectorSubcoreMesh(core_axis_name="c", subcore_axis_name="s",
                                  num_cores=2, num_subcores=NSUB)
    run = pl.kernel(
        _edge_kernel1,
        mesh=mesh,
        compiler_params=pltpu.CompilerParams(needs_layout_passes=False,
                                             use_tc_tiling_on_sc=False),
        out_type=jax.ShapeDtypeStruct((2, NPAD, 144), jnp.float32),
        scratch_types=[
            pltpu.VMEM_SHARED((NPAD, 144), jnp.float32),
            pltpu.VMEM((CHUNK,), jnp.int32),
            pltpu.VMEM((CHUNK,), jnp.int32),
            pltpu.VMEM((CHUNK,), jnp.int32),
            pltpu.VMEM((CHUNK,), jnp.int32),
            pltpu.VMEM((CHUNK,), jnp.int32),
            pltpu.VMEM((CHUNK,), jnp.int32),
            pltpu.VMEM((CHUNK, 8), jnp.float32),
            pltpu.VMEM((CHUNK, 8), jnp.float32),
            pltpu.VMEM((CHUNK, 8), jnp.float32),
            pltpu.VMEM((CHUNK, 8), jnp.float32),
            pltpu.VMEM((2 * CHUNK,), jnp.float32),
            pltpu.VMEM((CHUNK, 128), jnp.float32),
            pltpu.VMEM((CHUNK, 128), jnp.float32),
            pltpu.VMEM((CHUNK, 144), jnp.float32),
            pltpu.SemaphoreType.DMA,
            pltpu.SemaphoreType.DMA,
        ],
    )
    return run(src, dst, h2c, f)


def _edge_kernel2(src_hbm, dst_hbm, h_hbm, f_hbm, out_hbm,
                  acc_sh, f_v, sidx0, sidx1, didx0, didx1, w_v,
                  rows0, rows1, outb_v, semg0, semg1):
    c = lax.axis_index("c")
    s = lax.axis_index("s")
    sidx = (sidx0, sidx1)
    didx = (didx0, didx1)
    rows = (rows0, rows1)
    semg = (semg0, semg1)

    _zero_vmem_rows(outb_v, CHUNK, 5)
    for k in range(ROWS_PER_SUB // WCOPY):
        pltpu.sync_copy(outb_v.at[pl.ds(0, WCOPY)],
                        acc_sh.at[pl.ds(s * ROWS_PER_SUB + k * WCOPY, WCOPY)])
    plsc.subcore_barrier()

    pltpu.sync_copy(f_hbm, f_v)

    ep_w = EPAD // (2 * NSUB)
    base_t = c * (EPAD // 2) + s * ep_w
    nch = ep_w // CHUNK

    def fire(b, g):
        base = base_t + g * CHUNK
        pltpu.sync_copy(src_hbm.at[pl.ds(base, CHUNK)], sidx[b])
        pltpu.sync_copy(dst_hbm.at[pl.ds(base, CHUNK)], didx[b])
        pltpu.async_copy(h_hbm.at[didx[b]], rows[b], semg[b])

    def drain(b):
        pltpu.make_async_copy(h_hbm.at[didx[b]], rows[b], semg[b]).wait()

    def process(b):
        lanes = lax.iota(jnp.int32, 16)
        for blk in range(NBLK):
            sv = sidx[b][pl.ds(blk * 16, 16)]
            dv = didx[b][pl.ds(blk * 16, 16)]
            fs = plsc.load_gather(f_v, [sv * 2])
            fd = plsc.load_gather(f_v, [dv * 2 + 1])
            z = fs + fd
            w_v[pl.ds(blk * 16, 16)] = jnp.exp(-jnp.where(z >= 0, z, ALPHA * z))

        @plsc.parallel_loop(0, CHUNK, unroll=4)
        def _scale(e):
            wv = plsc.load_gather(w_v, [_splat_i32(e)])
            for gg in range(4):
                outb_v[e, pl.ds(gg * 16, 16)] = rows[b][e, pl.ds(gg * 16, 16)] * wv

        for blk in range(NBLK):
            ev = lanes + blk * 16
            plsc.store_scatter(outb_v, [ev, _splat_i32(64)],
                               w_v[pl.ds(blk * 16, 16)])
        pltpu.sync_copy(outb_v, acc_sh.at[sidx[b]], add=True)

    fire(0, 0)

    def outer(go, _):
        for b in range(2):
            g = go * 2 + b

            @pl.when(g + 1 < nch)
            def _():
                fire(1 - b, g + 1)
            drain(b)
            process(b)
        return 0
    lax.fori_loop(0, nch // 2, outer, 0)

    plsc.subcore_barrier()
    for k in range(ROWS_PER_SUB // WCOPY):
        r0 = s * ROWS_PER_SUB + k * WCOPY
        pltpu.sync_copy(acc_sh.at[pl.ds(r0, WCOPY)], out_hbm.at[c, pl.ds(r0, WCOPY)])


def _edge2_sc(src, dst, h2, f2_flat):
    mesh = plsc.VectorSubcoreMesh(core_axis_name="c", subcore_axis_name="s",
                                  num_cores=2, num_subcores=NSUB)
    run = pl.kernel(
        _edge_kernel2,
        mesh=mesh,
        compiler_params=pltpu.CompilerParams(needs_layout_passes=False,
                                             use_tc_tiling_on_sc=False),
        out_type=jax.ShapeDtypeStruct((2, NPAD, 80), jnp.float32),
        scratch_types=[
            pltpu.VMEM_SHARED((NPAD, 80), jnp.float32),
            pltpu.VMEM((NPAD * 2,), jnp.float32),
            pltpu.VMEM((CHUNK,), jnp.int32),
            pltpu.VMEM((CHUNK,), jnp.int32),
            pltpu.VMEM((CHUNK,), jnp.int32),
            pltpu.VMEM((CHUNK,), jnp.int32),
            pltpu.VMEM((CHUNK,), jnp.float32),
            pltpu.VMEM((CHUNK, 64), jnp.float32),
            pltpu.VMEM((CHUNK, 64), jnp.float32),
            pltpu.VMEM((CHUNK, 80), jnp.float32),
            pltpu.SemaphoreType.DMA,
            pltpu.SemaphoreType.DMA,
        ],
    )
    return run(src, dst, h2, f2_flat)


def kernel(adj, x, W, a, W_out, a_out):
    src = adj[0].astype(jnp.int32)
    dst = adj[1].astype(jnp.int32)

    wcat = jnp.transpose(W, (1, 0, 2)).reshape(NFEAT, NHEADS * NHID)
    a_src = a[:, :NHID]
    a_dst = a[:, NHID:]
    blk_s = jax.scipy.linalg.block_diag(*[a_src[i][:, None] for i in range(NHEADS)])
    blk_d = jax.scipy.linalg.block_diag(*[a_dst[i][:, None] for i in range(NHEADS)])
    wa = jnp.concatenate([wcat @ blk_s, wcat @ blk_d], axis=1)
    wa2 = jnp.concatenate(
        [(W_out @ a_out[:NCLASS])[:, None], (W_out @ a_out[NCLASS:])[:, None],
         jnp.zeros((NHEADS * NHID, 6), jnp.float32)], axis=1)

    xpad = jnp.pad(x, ((0, NPAD - N), (0, 0)))
    src_p = jnp.pad(src, (0, EPAD - src.shape[0]), constant_values=N)
    dst_p = jnp.pad(dst, (0, EPAD - dst.shape[0]), constant_values=0)

    h, f = _mm1(xpad, wcat, wa)
    acc1 = _edge1_sc(src_p, dst_p, h.reshape(2 * NPAD, 128), f)
    h2, f2 = _mm2(acc1, W_out, wa2)
    acc2 = _edge2_sc(src_p, dst_p, h2, f2[:, :2].reshape(-1))
    out = _fin(acc2)
    return out[:N]


# speedup vs baseline: 71.5250x; 1.5269x over previous
import functools

import jax
import jax.numpy as jnp
from jax import lax
from jax.experimental import pallas as pl
from jax.experimental.pallas import tpu as pltpu
from jax.experimental.pallas import tpu_sc as plsc

N = 10000
NFEAT = 128
NHID = 64
NCLASS = 64
NHEADS = 4
ALPHA = 0.2

BN = 400
H1W = 136
H2W = 72


def _elu(z):
    return jnp.where(z > 0, z, jnp.exp(z) - 1.0)


def _mm1_body(x_ref, w_ref, a_ref, h_ref, f_ref):
    c = pl.program_id(0)
    xb = x_ref[...]
    wc = jnp.concatenate([w_ref[2 * c], w_ref[2 * c + 1]], axis=1)
    h_ref[:, :128] = jnp.dot(xb, wc, preferred_element_type=jnp.float32)
    h_ref[:, 128:] = jnp.zeros((BN, H1W - 128), jnp.float32)
    cols = [jnp.dot(w_ref[j], a_ref[j, :NHID],
                    preferred_element_type=jnp.float32) for j in range(NHEADS)]
    cols += [jnp.dot(w_ref[j], a_ref[j, NHID:],
                     preferred_element_type=jnp.float32) for j in range(NHEADS)]
    wa = jnp.stack(cols, axis=1)
    f_ref[...] = jnp.dot(xb, wa, preferred_element_type=jnp.float32)


def _mm1(x, W, a):
    return pl.pallas_call(
        _mm1_body,
        grid=(2, N // BN),
        in_specs=[
            pl.BlockSpec((BN, NFEAT), lambda c, i: (i, 0)),
            pl.BlockSpec((NHEADS, NFEAT, NHID), lambda c, i: (0, 0, 0)),
            pl.BlockSpec((NHEADS, 2 * NHID), lambda c, i: (0, 0)),
        ],
        out_specs=[
            pl.BlockSpec((BN, H1W), lambda c, i: (c * (N // BN) + i, 0)),
            pl.BlockSpec((BN, 8), lambda c, i: (i, 0)),
        ],
        out_shape=[
            jax.ShapeDtypeStruct((2 * N, H1W), jnp.float32),
            jax.ShapeDtypeStruct((N, 8), jnp.float32),
        ],
    )(x, W, a)


def _mm2_body(acc_ref, wout_ref, aout_ref, h2_ref, f2_ref):
    parts = []
    for i in range(NHEADS):
        c, hh = divmod(i, 2)
        v = acc_ref[c, :, hh * 64:(hh + 1) * 64]
        rs = acc_ref[c, :, 128 + hh][:, None]
        parts.append(_elu(v / (rs + 1e-16)))
    x2b = jnp.concatenate(parts, axis=1)
    h2_ref[:, :64] = jnp.dot(x2b, wout_ref[...], preferred_element_type=jnp.float32)
    h2_ref[:, 64:] = jnp.zeros((BN, H2W - 64), jnp.float32)
    wa2 = jnp.stack(
        [jnp.dot(wout_ref[...], aout_ref[:NCLASS],
                 preferred_element_type=jnp.float32),
         jnp.dot(wout_ref[...], aout_ref[NCLASS:],
                 preferred_element_type=jnp.float32)]
        + [jnp.zeros((NHEADS * NHID,), jnp.float32)] * 6, axis=1)
    f2_ref[...] = jnp.dot(x2b, wa2, preferred_element_type=jnp.float32)


def _mm2(acc1, w_out, a_out):
    return pl.pallas_call(
        _mm2_body,
        grid=(N // BN,),
        in_specs=[
            pl.BlockSpec((2, BN, H1W), lambda i: (0, i, 0)),
            pl.BlockSpec((NHEADS * NHID, NCLASS), lambda i: (0, 0)),
            pl.BlockSpec((2 * NCLASS,), lambda i: (0,)),
        ],
        out_specs=[
            pl.BlockSpec((BN, H2W), lambda i: (i, 0)),
            pl.BlockSpec((BN, 8), lambda i: (i, 0)),
        ],
        out_shape=[
            jax.ShapeDtypeStruct((N, H2W), jnp.float32),
            jax.ShapeDtypeStruct((N, 8), jnp.float32),
        ],
    )(acc1, w_out, a_out)


def _fin_body(acc_ref, out_ref):
    s = acc_ref[0, :, :64] + acc_ref[1, :, :64]
    rs = (acc_ref[0, :, 64] + acc_ref[1, :, 64])[:, None]
    out_ref[...] = _elu(s / (rs + 1e-16))


def _fin(acc2):
    return pl.pallas_call(
        _fin_body,
        grid=(N // BN,),
        in_specs=[pl.BlockSpec((2, BN, H2W), lambda i: (0, i, 0))],
        out_specs=pl.BlockSpec((BN, NCLASS), lambda i: (i, 0)),
        out_shape=jax.ShapeDtypeStruct((N, NCLASS), jnp.float32),
    )(acc2)


NEDGE = 320000
CHUNK = 80
NSUB = 16
ROWS_PER_SUB = N // NSUB
NBLK = CHUNK // 16
WSPLIT = [(k * 80, 80) for k in range(7)] + [(560, 65)]


def _splat_i32(x):
    return jnp.zeros((16,), jnp.int32) + x


def _zero_acc(zbuf, acc_sh, s, width):
    offs = [g * 16 for g in range(width // 16)]
    if width % 16:
        offs.append(width - 16)

    def zrow(r, _):
        for off in offs:
            zbuf[r, pl.ds(off, 16)] = jnp.zeros((16,), jnp.float32)
        return 0
    lax.fori_loop(0, CHUNK, zrow, 0)
    for off, nr in WSPLIT:
        pltpu.sync_copy(zbuf.at[pl.ds(0, nr)],
                        acc_sh.at[pl.ds(s * ROWS_PER_SUB + off, nr)])
    plsc.subcore_barrier()


def _writeout(acc_sh, out_hbm, c, s):
    plsc.subcore_barrier()
    for off, nr in WSPLIT:
        r0 = s * ROWS_PER_SUB + off
        pltpu.sync_copy(acc_sh.at[pl.ds(r0, nr)], out_hbm.at[c, pl.ds(r0, nr)])


def _edge_kernel1(src_hbm, dst_hbm, h_hbm, f_hbm, out_hbm,
                  acc_sh, sidx0, sidx1, didx0, didx1, gidx0, gidx1,
                  fs0, fs1, fd0, fd1, w_v, rows0, rows1,
                  semg0, semg1, semi0, semi1, sems0, sems1):
    c = lax.axis_index("c")
    s = lax.axis_index("s")
    sidx = (sidx0, sidx1)
    didx = (didx0, didx1)
    gidx = (gidx0, gidx1)
    fsb = (fs0, fs1)
    fdb = (fd0, fd1)
    rows = (rows0, rows1)
    semg = (semg0, semg1)
    semi = (semi0, semi1)
    sems = (sems0, sems1)
    lanes = lax.iota(jnp.int32, 16)

    _zero_acc(rows0, acc_sh, s, H1W)

    ep_tile = NEDGE // NSUB
    base_t = s * ep_tile
    nch = ep_tile // CHUNK

    def idx_load(b, g):
        base = base_t + g * CHUNK
        pltpu.async_copy(src_hbm.at[pl.ds(base, CHUNK)], sidx[b], semi[b])
        pltpu.async_copy(dst_hbm.at[pl.ds(base, CHUNK)], didx[b], semi[b])

    def idx_drain(b):
        pltpu.make_async_copy(src_hbm.at[pl.ds(0, CHUNK)], sidx[b], semi[b]).wait()
        pltpu.make_async_copy(dst_hbm.at[pl.ds(0, CHUNK)], didx[b], semi[b]).wait()

    def fire_gather(b):
        for blk in range(NBLK):
            gidx[b][pl.ds(blk * 16, 16)] = didx[b][pl.ds(blk * 16, 16)] + c * N
        pltpu.async_copy(h_hbm.at[gidx[b]], rows[b], semg[b])
        pltpu.async_copy(f_hbm.at[sidx[b]], fsb[b], semg[b])
        pltpu.async_copy(f_hbm.at[didx[b]], fdb[b], semg[b])

    def gather_drain(b):
        pltpu.make_async_copy(h_hbm.at[gidx[b]], rows[b], semg[b]).wait()
        pltpu.make_async_copy(f_hbm.at[sidx[b]], fsb[b], semg[b]).wait()
        pltpu.make_async_copy(f_hbm.at[didx[b]], fdb[b], semg[b]).wait()

    def scatter_drain(b):
        pltpu.make_async_copy(rows[b], acc_sh.at[sidx[b]], sems[b]).wait()

    def process(b):
        for blk in range(NBLK):
            for hh in range(2):
                head = c * 2 + hh
                fs = plsc.load_gather(fsb[b], [lanes + blk * 16, _splat_i32(head)])
                fd = plsc.load_gather(fdb[b], [lanes + blk * 16, _splat_i32(head + 4)])
                z = fs + fd
                w = jnp.exp(-jnp.where(z >= 0, z, ALPHA * z))
                w_v[pl.ds(hh * CHUNK + blk * 16, 16)] = w

        @plsc.parallel_loop(0, CHUNK, unroll=4)
        def _scale(e):
            wv0 = plsc.load_gather(w_v, [_splat_i32(e)])
            wv1 = plsc.load_gather(w_v, [_splat_i32(e + CHUNK)])
            for gg in range(4):
                rows[b][e, pl.ds(gg * 16, 16)] = rows[b][e, pl.ds(gg * 16, 16)] * wv0
            for gg in range(4, 8):
                rows[b][e, pl.ds(gg * 16, 16)] = rows[b][e, pl.ds(gg * 16, 16)] * wv1

        for blk in range(NBLK):
            plsc.store_scatter(rows[b], [lanes + blk * 16, _splat_i32(128)],
                               w_v[pl.ds(blk * 16, 16)])
            plsc.store_scatter(rows[b], [lanes + blk * 16, _splat_i32(129)],
                               w_v[pl.ds(CHUNK + blk * 16, 16)])

    idx_load(0, 0)
    idx_drain(0)
    fire_gather(0)
    idx_load(1, 1)

    def outer(go, _):
        for b in range(2):
            g = go * 2 + b
            nb = 1 - b

            @pl.when(g + 1 < nch)
            def _():
                idx_drain(nb)

                @pl.when(g >= 1)
                def _():
                    scatter_drain(nb)
                fire_gather(nb)
            gather_drain(b)
            process(b)
            pltpu.async_copy(rows[b], acc_sh.at[sidx[b]], sems[b], add=True)

            @pl.when(g + 2 < nch)
            def _():
                idx_load(b, g + 2)
        return 0
    lax.fori_loop(0, nch // 2, outer, 0)
    scatter_drain(0)
    scatter_drain(1)

    _writeout(acc_sh, out_hbm, c, s)


def _edge1_sc(src, dst, haug, f):
    mesh = plsc.VectorSubcoreMesh(core_axis_name="c", subcore_axis_name="s",
                                  num_cores=2, num_subcores=NSUB)
    run = pl.kernel(
        _edge_kernel1,
        mesh=mesh,
        compiler_params=pltpu.CompilerParams(needs_layout_passes=False,
                                             use_tc_tiling_on_sc=False),
        out_type=jax.ShapeDtypeStruct((2, N, H1W), jnp.float32),
        scratch_types=[
            pltpu.VMEM_SHARED((N, H1W), jnp.float32),
            pltpu.VMEM((CHUNK,), jnp.int32),
            pltpu.VMEM((CHUNK,), jnp.int32),
            pltpu.VMEM((CHUNK,), jnp.int32),
            pltpu.VMEM((CHUNK,), jnp.int32),
            pltpu.VMEM((CHUNK,), jnp.int32),
            pltpu.VMEM((CHUNK,), jnp.int32),
            pltpu.VMEM((CHUNK, 8), jnp.float32),
            pltpu.VMEM((CHUNK, 8), jnp.float32),
            pltpu.VMEM((CHUNK, 8), jnp.float32),
            pltpu.VMEM((CHUNK, 8), jnp.float32),
            pltpu.VMEM((2 * CHUNK,), jnp.float32),
            pltpu.VMEM((CHUNK, H1W), jnp.float32),
            pltpu.VMEM((CHUNK, H1W), jnp.float32),
            pltpu.SemaphoreType.DMA,
            pltpu.SemaphoreType.DMA,
            pltpu.SemaphoreType.DMA,
            pltpu.SemaphoreType.DMA,
            pltpu.SemaphoreType.DMA,
            pltpu.SemaphoreType.DMA,
        ],
    )
    return run(src, dst, haug, f)


def _edge_kernel2(src_hbm, dst_hbm, h_hbm, f_hbm, out_hbm,
                  acc_sh, f_v, sidx0, sidx1, didx0, didx1, w_v,
                  rows0, rows1, semg0, semg1, semi0, semi1, sems0, sems1):
    c = lax.axis_index("c")
    s = lax.axis_index("s")
    sidx = (sidx0, sidx1)
    didx = (didx0, didx1)
    rows = (rows0, rows1)
    semg = (semg0, semg1)
    semi = (semi0, semi1)
    sems = (sems0, sems1)
    lanes = lax.iota(jnp.int32, 16)

    _zero_acc(rows0, acc_sh, s, H2W)

    pltpu.sync_copy(f_hbm, f_v)

    ep_w = NEDGE // (2 * NSUB)
    base_t = c * (NEDGE // 2) + s * ep_w
    nch = ep_w // CHUNK

    def idx_load(b, g):
        base = base_t + g * CHUNK
        pltpu.async_copy(src_hbm.at[pl.ds(base, CHUNK)], sidx[b], semi[b])
        pltpu.async_copy(dst_hbm.at[pl.ds(base, CHUNK)], didx[b], semi[b])

    def idx_drain(b):
        pltpu.make_async_copy(src_hbm.at[pl.ds(0, CHUNK)], sidx[b], semi[b]).wait()
        pltpu.make_async_copy(dst_hbm.at[pl.ds(0, CHUNK)], didx[b], semi[b]).wait()

    def fire_gather(b):
        pltpu.async_copy(h_hbm.at[didx[b]], rows[b], semg[b])

    def gather_drain(b):
        pltpu.make_async_copy(h_hbm.at[didx[b]], rows[b], semg[b]).wait()

    def scatter_drain(b):
        pltpu.make_async_copy(rows[b], acc_sh.at[sidx[b]], sems[b]).wait()

    def process(b):
        for blk in range(NBLK):
            sv = sidx[b][pl.ds(blk * 16, 16)]
            dv = didx[b][pl.ds(blk * 16, 16)]
            fs = plsc.load_gather(f_v, [sv * 2])
            fd = plsc.load_gather(f_v, [dv * 2 + 1])
            z = fs + fd
            w_v[pl.ds(blk * 16, 16)] = jnp.exp(-jnp.where(z >= 0, z, ALPHA * z))

        @plsc.parallel_loop(0, CHUNK, unroll=4)
        def _scale(e):
            wv = plsc.load_gather(w_v, [_splat_i32(e)])
            for gg in range(4):
                rows[b][e, pl.ds(gg * 16, 16)] = rows[b][e, pl.ds(gg * 16, 16)] * wv

        for blk in range(NBLK):
            plsc.store_scatter(rows[b], [lanes + blk * 16, _splat_i32(64)],
                               w_v[pl.ds(blk * 16, 16)])

    idx_load(0, 0)
    idx_drain(0)
    fire_gather(0)
    idx_load(1, 1)

    def outer(go, _):
        for b in range(2):
            g = go * 2 + b
            nb = 1 - b

            @pl.when(g + 1 < nch)
            def _():
                idx_drain(nb)

                @pl.when(g >= 1)
                def _():
                    scatter_drain(nb)
                fire_gather(nb)
            gather_drain(b)
            process(b)
            pltpu.async_copy(rows[b], acc_sh.at[sidx[b]], sems[b], add=True)

            @pl.when(g + 2 < nch)
            def _():
                idx_load(b, g + 2)
        return 0
    lax.fori_loop(0, nch // 2, outer, 0)
    eb = (nch - 1) % 2
    gather_drain(eb)
    process(eb)
    pltpu.async_copy(rows[eb], acc_sh.at[sidx[eb]], sems[eb], add=True)
    scatter_drain(1 - eb)
    scatter_drain(eb)

    _writeout(acc_sh, out_hbm, c, s)


def _edge2_sc(src, dst, h2aug, f2_flat):
    mesh = plsc.VectorSubcoreMesh(core_axis_name="c", subcore_axis_name="s",
                                  num_cores=2, num_subcores=NSUB)
    run = pl.kernel(
        _edge_kernel2,
        mesh=mesh,
        compiler_params=pltpu.CompilerParams(needs_layout_passes=False,
                                             use_tc_tiling_on_sc=False),
        out_type=jax.ShapeDtypeStruct((2, N, H2W), jnp.float32),
        scratch_types=[
            pltpu.VMEM_SHARED((N, H2W), jnp.float32),
            pltpu.VMEM((2 * N,), jnp.float32),
            pltpu.VMEM((CHUNK,), jnp.int32),
            pltpu.VMEM((CHUNK,), jnp.int32),
            pltpu.VMEM((CHUNK,), jnp.int32),
            pltpu.VMEM((CHUNK,), jnp.int32),
            pltpu.VMEM((CHUNK,), jnp.float32),
            pltpu.VMEM((CHUNK, H2W), jnp.float32),
            pltpu.VMEM((CHUNK, H2W), jnp.float32),
            pltpu.SemaphoreType.DMA,
            pltpu.SemaphoreType.DMA,
            pltpu.SemaphoreType.DMA,
            pltpu.SemaphoreType.DMA,
            pltpu.SemaphoreType.DMA,
            pltpu.SemaphoreType.DMA,
        ],
    )
    return run(src, dst, h2aug, f2_flat)


def kernel(adj, x, W, a, W_out, a_out):
    src = adj[0].astype(jnp.int32)
    dst = adj[1].astype(jnp.int32)

    h, f = _mm1(x, W, a)
    acc1 = _edge1_sc(src, dst, h, f)
    h2, f2 = _mm2(acc1, W_out, a_out)
    acc2 = _edge2_sc(src, dst, h2, f2[:, :2].reshape(-1))
    return _fin(acc2)


# speedup vs baseline: 78.1975x; 1.0933x over previous
import functools

import jax
import jax.numpy as jnp
from jax import lax
from jax.experimental import pallas as pl
from jax.experimental.pallas import tpu as pltpu
from jax.experimental.pallas import tpu_sc as plsc

N = 10000
NFEAT = 128
NHID = 64
NCLASS = 64
NHEADS = 4
ALPHA = 0.2

BN = 2000
H1W = 136
H2W = 72


def _elu(z):
    return jnp.where(z > 0, z, jnp.exp(z) - 1.0)


def _mm1_body(x_ref, w_ref, a_ref, h_ref, f_ref):
    c = pl.program_id(0)
    xb = x_ref[...]
    cols = [jnp.dot(w_ref[j], a_ref[j, :NHID],
                    preferred_element_type=jnp.float32) for j in range(NHEADS)]
    cols += [jnp.dot(w_ref[j], a_ref[j, NHID:],
                     preferred_element_type=jnp.float32) for j in range(NHEADS)]
    wfull = jnp.concatenate(
        [w_ref[2 * c], w_ref[2 * c + 1], jnp.stack(cols, axis=1)], axis=1)
    hb = jnp.dot(xb, wfull, preferred_element_type=jnp.float32)
    h_ref[...] = hb
    f_ref[...] = hb[:, 128:]


def _mm1(x, W, a):
    return pl.pallas_call(
        _mm1_body,
        grid=(2, N // BN),
        in_specs=[
            pl.BlockSpec((BN, NFEAT), lambda c, i: (i, 0)),
            pl.BlockSpec((NHEADS, NFEAT, NHID), lambda c, i: (0, 0, 0)),
            pl.BlockSpec((NHEADS, 2 * NHID), lambda c, i: (0, 0)),
        ],
        out_specs=[
            pl.BlockSpec((BN, H1W), lambda c, i: (c * (N // BN) + i, 0)),
            pl.BlockSpec((BN, 8), lambda c, i: (i, 0)),
        ],
        out_shape=[
            jax.ShapeDtypeStruct((2 * N, H1W), jnp.float32),
            jax.ShapeDtypeStruct((N, 8), jnp.float32),
        ],
    )(x, W, a)


def _mm2_body(acc_ref, wout_ref, aout_ref, h2_ref, f2_ref):
    parts = []
    for i in range(NHEADS):
        c, hh = divmod(i, 2)
        v = acc_ref[c, :, hh * 64:(hh + 1) * 64]
        rs = acc_ref[c, :, 128 + hh][:, None]
        parts.append(_elu(v / (rs + 1e-16)))
    x2b = jnp.concatenate(parts, axis=1)
    h2_ref[:, :64] = jnp.dot(x2b, wout_ref[...], preferred_element_type=jnp.float32)
    h2_ref[:, 64:] = jnp.zeros((BN, H2W - 64), jnp.float32)
    wa2 = jnp.stack(
        [jnp.dot(wout_ref[...], aout_ref[:NCLASS],
                 preferred_element_type=jnp.float32),
         jnp.dot(wout_ref[...], aout_ref[NCLASS:],
                 preferred_element_type=jnp.float32)]
        + [jnp.zeros((NHEADS * NHID,), jnp.float32)] * 6, axis=1)
    f2_ref[...] = jnp.dot(x2b, wa2, preferred_element_type=jnp.float32)


def _mm2(acc1, w_out, a_out):
    return pl.pallas_call(
        _mm2_body,
        grid=(N // BN,),
        in_specs=[
            pl.BlockSpec((2, BN, H1W), lambda i: (0, i, 0)),
            pl.BlockSpec((NHEADS * NHID, NCLASS), lambda i: (0, 0)),
            pl.BlockSpec((2 * NCLASS,), lambda i: (0,)),
        ],
        out_specs=[
            pl.BlockSpec((BN, H2W), lambda i: (i, 0)),
            pl.BlockSpec((BN, 8), lambda i: (i, 0)),
        ],
        out_shape=[
            jax.ShapeDtypeStruct((N, H2W), jnp.float32),
            jax.ShapeDtypeStruct((N, 8), jnp.float32),
        ],
    )(acc1, w_out, a_out)


def _fin_body(acc_ref, out_ref):
    s = acc_ref[0, :, :64] + acc_ref[1, :, :64]
    rs = (acc_ref[0, :, 64] + acc_ref[1, :, 64])[:, None]
    out_ref[...] = _elu(s / (rs + 1e-16))


def _fin(acc2):
    return pl.pallas_call(
        _fin_body,
        grid=(N // BN,),
        in_specs=[pl.BlockSpec((2, BN, H2W), lambda i: (0, i, 0))],
        out_specs=pl.BlockSpec((BN, NCLASS), lambda i: (i, 0)),
        out_shape=jax.ShapeDtypeStruct((N, NCLASS), jnp.float32),
    )(acc2)


NEDGE = 320000
CHUNK = 80
NSUB = 16
ROWS_PER_SUB = N // NSUB
NBLK = CHUNK // 16
WSPLIT = [(k * 80, 80) for k in range(7)] + [(560, 65)]


def _splat_i32(x):
    return jnp.zeros((16,), jnp.int32) + x


def _zero_acc(zbuf, acc_sh, s, width):
    offs = [g * 16 for g in range(width // 16)]
    if width % 16:
        offs.append(width - 16)

    def zrow(r, _):
        for off in offs:
            zbuf[r, pl.ds(off, 16)] = jnp.zeros((16,), jnp.float32)
        return 0
    lax.fori_loop(0, CHUNK, zrow, 0)
    for off, nr in WSPLIT:
        pltpu.sync_copy(zbuf.at[pl.ds(0, nr)],
                        acc_sh.at[pl.ds(s * ROWS_PER_SUB + off, nr)])
    plsc.subcore_barrier()


def _writeout(acc_sh, out_hbm, c, s):
    plsc.subcore_barrier()
    for off, nr in WSPLIT:
        r0 = s * ROWS_PER_SUB + off
        pltpu.sync_copy(acc_sh.at[pl.ds(r0, nr)], out_hbm.at[c, pl.ds(r0, nr)])


def _edge_kernel1(adj_hbm, h_hbm, f_hbm, out_hbm,
                  acc_sh, sidx0, sidx1, didx0, didx1, gidx0, gidx1,
                  fs0, fs1, fd0, fd1, w_v, rows0, rows1,
                  semg0, semg1, semi0, semi1, sems0, sems1):
    c = lax.axis_index("c")
    s = lax.axis_index("s")
    sidx = (sidx0, sidx1)
    didx = (didx0, didx1)
    gidx = (gidx0, gidx1)
    fsb = (fs0, fs1)
    fdb = (fd0, fd1)
    rows = (rows0, rows1)
    semg = (semg0, semg1)
    semi = (semi0, semi1)
    sems = (sems0, sems1)
    lanes = lax.iota(jnp.int32, 16)

    _zero_acc(rows0, acc_sh, s, H1W)

    ep_tile = NEDGE // NSUB
    base_t = s * ep_tile
    nch = ep_tile // CHUNK

    def idx_load(b, g):
        base = base_t + g * CHUNK
        pltpu.async_copy(adj_hbm.at[0, pl.ds(base, CHUNK)], sidx[b], semi[b])
        pltpu.async_copy(adj_hbm.at[1, pl.ds(base, CHUNK)], didx[b], semi[b])

    def idx_drain(b):
        pltpu.make_async_copy(adj_hbm.at[0, pl.ds(0, CHUNK)], sidx[b], semi[b]).wait()
        pltpu.make_async_copy(adj_hbm.at[1, pl.ds(0, CHUNK)], didx[b], semi[b]).wait()

    def fire_gather(b):
        for blk in range(NBLK):
            gidx[b][pl.ds(blk * 16, 16)] = didx[b][pl.ds(blk * 16, 16)] + c * N
        pltpu.async_copy(h_hbm.at[gidx[b]], rows[b], semg[b])
        pltpu.async_copy(f_hbm.at[sidx[b]], fsb[b], semg[b])
        pltpu.async_copy(f_hbm.at[didx[b]], fdb[b], semg[b])

    def gather_drain(b):
        pltpu.make_async_copy(h_hbm.at[gidx[b]], rows[b], semg[b]).wait()
        pltpu.make_async_copy(f_hbm.at[sidx[b]], fsb[b], semg[b]).wait()
        pltpu.make_async_copy(f_hbm.at[didx[b]], fdb[b], semg[b]).wait()

    def scatter_drain(b):
        pltpu.make_async_copy(rows[b], acc_sh.at[sidx[b]], sems[b]).wait()

    def process(b):
        for blk in range(NBLK):
            for hh in range(2):
                head = c * 2 + hh
                fs = plsc.load_gather(fsb[b], [lanes + blk * 16, _splat_i32(head)])
                fd = plsc.load_gather(fdb[b], [lanes + blk * 16, _splat_i32(head + 4)])
                z = fs + fd
                w = jnp.exp(-jnp.where(z >= 0, z, ALPHA * z))
                w_v[pl.ds(hh * CHUNK + blk * 16, 16)] = w

        @plsc.parallel_loop(0, CHUNK, unroll=4)
        def _scale(e):
            wv0 = plsc.load_gather(w_v, [_splat_i32(e)])
            wv1 = plsc.load_gather(w_v, [_splat_i32(e + CHUNK)])
            for gg in range(4):
                rows[b][e, pl.ds(gg * 16, 16)] = rows[b][e, pl.ds(gg * 16, 16)] * wv0
            for gg in range(4, 8):
                rows[b][e, pl.ds(gg * 16, 16)] = rows[b][e, pl.ds(gg * 16, 16)] * wv1

        for blk in range(NBLK):
            plsc.store_scatter(rows[b], [lanes + blk * 16, _splat_i32(128)],
                               w_v[pl.ds(blk * 16, 16)])
            plsc.store_scatter(rows[b], [lanes + blk * 16, _splat_i32(129)],
                               w_v[pl.ds(CHUNK + blk * 16, 16)])

    idx_load(0, 0)
    idx_drain(0)
    fire_gather(0)
    idx_load(1, 1)

    def outer(go, _):
        for b in range(2):
            g = go * 2 + b
            nb = 1 - b

            @pl.when(g + 1 < nch)
            def _():
                idx_drain(nb)

                @pl.when(g >= 1)
                def _():
                    scatter_drain(nb)
                fire_gather(nb)
            gather_drain(b)
            process(b)
            pltpu.async_copy(rows[b], acc_sh.at[sidx[b]], sems[b], add=True)

            @pl.when(g + 2 < nch)
            def _():
                idx_load(b, g + 2)
        return 0
    lax.fori_loop(0, nch // 2, outer, 0)
    scatter_drain(0)
    scatter_drain(1)

    _writeout(acc_sh, out_hbm, c, s)


def _edge1_sc(adj2, haug, f):
    mesh = plsc.VectorSubcoreMesh(core_axis_name="c", subcore_axis_name="s",
                                  num_cores=2, num_subcores=NSUB)
    run = pl.kernel(
        _edge_kernel1,
        mesh=mesh,
        compiler_params=pltpu.CompilerParams(needs_layout_passes=False,
                                             use_tc_tiling_on_sc=False),
        out_type=jax.ShapeDtypeStruct((2, N, H1W), jnp.float32),
        scratch_types=[
            pltpu.VMEM_SHARED((N, H1W), jnp.float32),
            pltpu.VMEM((CHUNK,), jnp.int32),
            pltpu.VMEM((CHUNK,), jnp.int32),
            pltpu.VMEM((CHUNK,), jnp.int32),
            pltpu.VMEM((CHUNK,), jnp.int32),
            pltpu.VMEM((CHUNK,), jnp.int32),
            pltpu.VMEM((CHUNK,), jnp.int32),
            pltpu.VMEM((CHUNK, 8), jnp.float32),
            pltpu.VMEM((CHUNK, 8), jnp.float32),
            pltpu.VMEM((CHUNK, 8), jnp.float32),
            pltpu.VMEM((CHUNK, 8), jnp.float32),
            pltpu.VMEM((2 * CHUNK,), jnp.float32),
            pltpu.VMEM((CHUNK, H1W), jnp.float32),
            pltpu.VMEM((CHUNK, H1W), jnp.float32),
            pltpu.SemaphoreType.DMA,
            pltpu.SemaphoreType.DMA,
            pltpu.SemaphoreType.DMA,
            pltpu.SemaphoreType.DMA,
            pltpu.SemaphoreType.DMA,
            pltpu.SemaphoreType.DMA,
        ],
    )
    return run(adj2, haug, f)


def _edge_kernel2(adj_hbm, h_hbm, f_hbm, out_hbm,
                  acc_sh, f_v, sidx0, sidx1, didx0, didx1, w_v,
                  rows0, rows1, semg0, semg1, semi0, semi1, sems0, sems1):
    c = lax.axis_index("c")
    s = lax.axis_index("s")
    sidx = (sidx0, sidx1)
    didx = (didx0, didx1)
    rows = (rows0, rows1)
    semg = (semg0, semg1)
    semi = (semi0, semi1)
    sems = (sems0, sems1)
    lanes = lax.iota(jnp.int32, 16)

    _zero_acc(rows0, acc_sh, s, H2W)

    pltpu.sync_copy(f_hbm, f_v)

    ep_w = NEDGE // (2 * NSUB)
    base_t = c * (NEDGE // 2) + s * ep_w
    nch = ep_w // CHUNK

    def idx_load(b, g):
        base = base_t + g * CHUNK
        pltpu.async_copy(adj_hbm.at[0, pl.ds(base, CHUNK)], sidx[b], semi[b])
        pltpu.async_copy(adj_hbm.at[1, pl.ds(base, CHUNK)], didx[b], semi[b])

    def idx_drain(b):
        pltpu.make_async_copy(adj_hbm.at[0, pl.ds(0, CHUNK)], sidx[b], semi[b]).wait()
        pltpu.make_async_copy(adj_hbm.at[1, pl.ds(0, CHUNK)], didx[b], semi[b]).wait()

    def fire_gather(b):
        pltpu.async_copy(h_hbm.at[didx[b]], rows[b], semg[b])

    def gather_drain(b):
        pltpu.make_async_copy(h_hbm.at[didx[b]], rows[b], semg[b]).wait()

    def scatter_drain(b):
        pltpu.make_async_copy(rows[b], acc_sh.at[sidx[b]], sems[b]).wait()

    def process(b):
        for blk in range(NBLK):
            sv = sidx[b][pl.ds(blk * 16, 16)]
            dv = didx[b][pl.ds(blk * 16, 16)]
            fs = plsc.load_gather(f_v, [sv * 2])
            fd = plsc.load_gather(f_v, [dv * 2 + 1])
            z = fs + fd
            w_v[pl.ds(blk * 16, 16)] = jnp.exp(-jnp.where(z >= 0, z, ALPHA * z))

        @plsc.parallel_loop(0, CHUNK, unroll=4)
        def _scale(e):
            wv = plsc.load_gather(w_v, [_splat_i32(e)])
            for gg in range(4):
                rows[b][e, pl.ds(gg * 16, 16)] = rows[b][e, pl.ds(gg * 16, 16)] * wv

        for blk in range(NBLK):
            plsc.store_scatter(rows[b], [lanes + blk * 16, _splat_i32(64)],
                               w_v[pl.ds(blk * 16, 16)])

    idx_load(0, 0)
    idx_drain(0)
    fire_gather(0)
    idx_load(1, 1)

    def outer(go, _):
        for b in range(2):
            g = go * 2 + b
            nb = 1 - b

            @pl.when(g + 1 < nch)
            def _():
                idx_drain(nb)

                @pl.when(g >= 1)
                def _():
                    scatter_drain(nb)
                fire_gather(nb)
            gather_drain(b)
            process(b)
            pltpu.async_copy(rows[b], acc_sh.at[sidx[b]], sems[b], add=True)

            @pl.when(g + 2 < nch)
            def _():
                idx_load(b, g + 2)
        return 0
    lax.fori_loop(0, nch // 2, outer, 0)
    eb = (nch - 1) % 2
    gather_drain(eb)
    process(eb)
    pltpu.async_copy(rows[eb], acc_sh.at[sidx[eb]], sems[eb], add=True)
    scatter_drain(1 - eb)
    scatter_drain(eb)

    _writeout(acc_sh, out_hbm, c, s)


def _edge2_sc(adj2, h2aug, f2_flat):
    mesh = plsc.VectorSubcoreMesh(core_axis_name="c", subcore_axis_name="s",
                                  num_cores=2, num_subcores=NSUB)
    run = pl.kernel(
        _edge_kernel2,
        mesh=mesh,
        compiler_params=pltpu.CompilerParams(needs_layout_passes=False,
                                             use_tc_tiling_on_sc=False),
        out_type=jax.ShapeDtypeStruct((2, N, H2W), jnp.float32),
        scratch_types=[
            pltpu.VMEM_SHARED((N, H2W), jnp.float32),
            pltpu.VMEM((2 * N,), jnp.float32),
            pltpu.VMEM((CHUNK,), jnp.int32),
            pltpu.VMEM((CHUNK,), jnp.int32),
            pltpu.VMEM((CHUNK,), jnp.int32),
            pltpu.VMEM((CHUNK,), jnp.int32),
            pltpu.VMEM((CHUNK,), jnp.float32),
            pltpu.VMEM((CHUNK, H2W), jnp.float32),
            pltpu.VMEM((CHUNK, H2W), jnp.float32),
            pltpu.SemaphoreType.DMA,
            pltpu.SemaphoreType.DMA,
            pltpu.SemaphoreType.DMA,
            pltpu.SemaphoreType.DMA,
            pltpu.SemaphoreType.DMA,
            pltpu.SemaphoreType.DMA,
        ],
    )
    return run(adj2, h2aug, f2_flat)


def kernel(adj, x, W, a, W_out, a_out):
    adj2 = adj.astype(jnp.int32)

    h, f = _mm1(x, W, a)
    acc1 = _edge1_sc(adj2, h, f)
    h2, f2 = _mm2(acc1, W_out, a_out)
    acc2 = _edge2_sc(adj2, h2, f2[:, :2].reshape(-1))
    return _fin(acc2)
